# baseline (speedup 1.0000x reference)
import functools

import numpy as np
import jax
import jax.numpy as jnp
from jax import lax
from jax.experimental import pallas as pl
from jax.experimental.pallas import tpu as pltpu
from jax.experimental.pallas import tpu_sc as plsc

NC = 2
NS = 16
L = 16


@functools.lru_cache(maxsize=None)
def _pos_encoding_np(length: int, dim: int) -> np.ndarray:
    pos = np.arange(length, dtype=np.float64)[:, None]
    i = np.arange(dim, dtype=np.float64)[None, :]
    angle_rates = 1.0 / np.power(10000.0, (2.0 * np.floor(i / 2.0)) / dim)
    angles = pos * angle_rates
    pe = np.where((np.arange(dim) % 2) == 0, np.sin(angles), np.cos(angles))
    return pe.astype(np.float32)


def kernel(x, table):
    B, T = x.shape
    V, D = table.shape
    NW = NC * NS
    POS_W = T // NW
    CH = 32
    NCH = POS_W // CH
    assert T % NW == 0 and POS_W % CH == 0 and D % L == 0

    pe = jnp.asarray(_pos_encoding_np(T, D))
    xf = x.reshape(B * T).astype(jnp.int32)

    mesh = plsc.VectorSubcoreMesh(
        core_axis_name="c", subcore_axis_name="s",
        num_cores=NC, num_subcores=NS)

    @functools.partial(
        pl.kernel,
        out_type=jax.ShapeDtypeStruct((B * T, D), jnp.float32),
        mesh=mesh,
        scratch_types=[
            pltpu.VMEM((B * POS_W,), jnp.int32),
            pltpu.VMEM((CH, D), jnp.float32),
            pltpu.VMEM((CH, D), jnp.float32),
            pltpu.SemaphoreType.DMA,
        ],
    )
    def sc_embed(x_hbm, pe_hbm, table_hbm, out_hbm, idx_v, pe_v, row_v, sem):
        wid = lax.axis_index("s") * NC + lax.axis_index("c")
        pbase = wid * POS_W
        for b in range(B):
            pltpu.sync_copy(x_hbm.at[pl.ds(b * T + pbase, POS_W)],
                            idx_v.at[pl.ds(b * POS_W, POS_W)])

        def chunk_body(c, carry):
            p0 = pbase + c * CH
            pltpu.sync_copy(pe_hbm.at[pl.ds(p0, CH)], pe_v)
            for b in range(B):
                idx_ref = idx_v.at[pl.ds(b * POS_W + c * CH, CH)]
                pltpu.async_copy(table_hbm.at[idx_ref], row_v, sem).wait()

                def add_row(r, carry2):
                    for dd in range(D // L):
                        sl = pl.ds(dd * L, L)
                        row_v[r, sl] = row_v[r, sl] + pe_v[r, sl]
                    return carry2

                lax.fori_loop(0, CH, add_row, None)
                pltpu.sync_copy(row_v, out_hbm.at[pl.ds(b * T + p0, CH)])
            return carry

        lax.fori_loop(0, NCH, chunk_body, None)

    out = sc_embed(xf, pe, table)
    return out.reshape(B, T, D)


# speedup vs baseline: 1.6352x; 1.6352x over previous
import functools

import numpy as np
import jax
import jax.numpy as jnp
from jax import lax
from jax.experimental import pallas as pl
from jax.experimental.pallas import tpu as pltpu
from jax.experimental.pallas import tpu_sc as plsc

NC = 2
NS = 16
L = 16
R = 4
A = 2


@functools.lru_cache(maxsize=None)
def _pos_encoding_np(length: int, dim: int) -> np.ndarray:
    pos = np.arange(length, dtype=np.float64)[:, None]
    i = np.arange(dim, dtype=np.float64)[None, :]
    angle_rates = 1.0 / np.power(10000.0, (2.0 * np.floor(i / 2.0)) / dim)
    angles = pos * angle_rates
    pe = np.where((np.arange(dim) % 2) == 0, np.sin(angles), np.cos(angles))
    return pe.astype(np.float32)


def kernel(x, table):
    B, T = x.shape
    V, D = table.shape
    NW = NC * NS
    POS_W = T // NW
    CH = 16
    NCH = POS_W // CH
    assert T % NW == 0 and POS_W % CH == 0 and D % L == 0

    pe = jnp.asarray(_pos_encoding_np(T, D))
    xf = x.reshape(B * T).astype(jnp.int32)

    mesh = plsc.VectorSubcoreMesh(
        core_axis_name="c", subcore_axis_name="s",
        num_cores=NC, num_subcores=NS)

    @functools.partial(
        pl.kernel,
        out_type=jax.ShapeDtypeStruct((B * T, D), jnp.float32),
        mesh=mesh,
        scratch_types=(
            [pltpu.VMEM((B * POS_W,), jnp.int32)]
            + [pltpu.VMEM((CH, D), jnp.float32) for _ in range(2 + R)]
            + [pltpu.SemaphoreType.DMA for _ in range(2 + 2 * R)]
        ),
    )
    def sc_embed(x_hbm, pe_hbm, table_hbm, out_hbm, idx_v,
                 pe0, pe1, r0, r1, r2, r3,
                 sp0, sp1, sg0, sg1, sg2, sg3, ss0, ss1, ss2, ss3):
        pe_bufs, pe_sems = [pe0, pe1], [sp0, sp1]
        row_bufs = [r0, r1, r2, r3]
        g_sems = [sg0, sg1, sg2, sg3]
        s_sems = [ss0, ss1, ss2, ss3]

        wid = lax.axis_index("s") * NC + lax.axis_index("c")
        pbase = wid * POS_W
        for b in range(B):
            pltpu.sync_copy(x_hbm.at[pl.ds(b * T + pbase, POS_W)],
                            idx_v.at[pl.ds(b * POS_W, POS_W)])

        items = [(c, b) for c in range(NCH) for b in range(B)]
        n_items = len(items)
        pe_desc, gather_desc, store_desc = {}, {}, {}

        def start_pe(c):
            pe_desc[c] = pltpu.async_copy(
                pe_hbm.at[pl.ds(pbase + c * CH, CH)],
                pe_bufs[c % 2], pe_sems[c % 2])

        def start_gather(i):
            if i - R in store_desc:
                store_desc.pop(i - R).wait()
            c, b = items[i]
            idx_ref = idx_v.at[pl.ds(b * POS_W + c * CH, CH)]
            gather_desc[i] = pltpu.async_copy(
                table_hbm.at[idx_ref], row_bufs[i % R], g_sems[i % R])

        start_pe(0)
        for i in range(min(A, n_items)):
            start_gather(i)

        for i in range(n_items):
            c, b = items[i]
            if b == 0:
                if c + 1 < NCH:
                    start_pe(c + 1)
                pe_desc.pop(c).wait()
            gather_desc.pop(i).wait()
            if i + A < n_items:
                start_gather(i + A)

            buf, pe_buf = row_bufs[i % R], pe_bufs[c % 2]

            def add_row(r, carry, buf=buf, pe_buf=pe_buf):
                for dd in range(D // L):
                    sl = pl.ds(dd * L, L)
                    plsc.addupdate(buf.at[r, sl], pe_buf[r, sl])
                return carry

            lax.fori_loop(0, CH, add_row, None)
            store_desc[i] = pltpu.async_copy(
                buf, out_hbm.at[pl.ds(b * T + pbase + c * CH, CH)],
                s_sems[i % R])

        for j in sorted(store_desc):
            store_desc.pop(j).wait()

    out = sc_embed(xf, pe, table)
    return out.reshape(B, T, D)


# speedup vs baseline: 1.6791x; 1.0269x over previous
import functools

import numpy as np
import jax
import jax.numpy as jnp
from jax import lax
from jax.experimental import pallas as pl
from jax.experimental.pallas import tpu as pltpu
from jax.experimental.pallas import tpu_sc as plsc

NC = 2
NS = 16
L = 16
R = 5
A = 3


@functools.lru_cache(maxsize=None)
def _pos_encoding_np(length: int, dim: int) -> np.ndarray:
    pos = np.arange(length, dtype=np.float64)[:, None]
    i = np.arange(dim, dtype=np.float64)[None, :]
    angle_rates = 1.0 / np.power(10000.0, (2.0 * np.floor(i / 2.0)) / dim)
    angles = pos * angle_rates
    pe = np.where((np.arange(dim) % 2) == 0, np.sin(angles), np.cos(angles))
    return pe.astype(np.float32)


def kernel(x, table):
    B, T = x.shape
    V, D = table.shape
    NW = NC * NS
    POS_W = T // NW
    CH = 16
    NCH = POS_W // CH
    assert T % NW == 0 and POS_W % CH == 0 and D % L == 0

    pe = jnp.asarray(_pos_encoding_np(T, D))
    xf = x.reshape(B * T).astype(jnp.int32)

    mesh = plsc.VectorSubcoreMesh(
        core_axis_name="c", subcore_axis_name="s",
        num_cores=NC, num_subcores=NS)

    @functools.partial(
        pl.kernel,
        out_type=jax.ShapeDtypeStruct((B * T, D), jnp.float32),
        mesh=mesh,
        scratch_types=(
            [pltpu.VMEM((B * POS_W,), jnp.int32)]
            + [pltpu.VMEM((CH, D), jnp.float32) for _ in range(2 + R)]
            + [pltpu.SemaphoreType.DMA for _ in range(3 + 2 * R)]
        ),
    )
    def sc_embed(x_hbm, pe_hbm, table_hbm, out_hbm, idx_v,
                 pe0, pe1, r0, r1, r2, r3, r4,
                 si, sp0, sp1, sg0, sg1, sg2, sg3, sg4,
                 ss0, ss1, ss2, ss3, ss4):
        pe_bufs, pe_sems = [pe0, pe1], [sp0, sp1]
        row_bufs = [r0, r1, r2, r3, r4]
        g_sems = [sg0, sg1, sg2, sg3, sg4]
        s_sems = [ss0, ss1, ss2, ss3, ss4]

        wid = lax.axis_index("s") * NC + lax.axis_index("c")
        pbase = wid * POS_W
        idx_descs = [
            pltpu.async_copy(x_hbm.at[pl.ds(b * T + pbase, POS_W)],
                             idx_v.at[pl.ds(b * POS_W, POS_W)], si)
            for b in range(B)]
        for dsc in idx_descs:
            dsc.wait()

        items = [(c, b) for c in range(NCH) for b in range(B)]
        n_items = len(items)
        pe_desc, gather_desc, store_desc = {}, {}, {}

        def start_pe(c):
            pe_desc[c] = pltpu.async_copy(
                pe_hbm.at[pl.ds(pbase + c * CH, CH)],
                pe_bufs[c % 2], pe_sems[c % 2])

        def start_gather(i):
            if i - R in store_desc:
                store_desc.pop(i - R).wait()
            c, b = items[i]
            idx_ref = idx_v.at[pl.ds(b * POS_W + c * CH, CH)]
            gather_desc[i] = pltpu.async_copy(
                table_hbm.at[idx_ref], row_bufs[i % R], g_sems[i % R])

        start_pe(0)
        for i in range(min(A, n_items)):
            start_gather(i)

        for i in range(n_items):
            c, b = items[i]
            if b == 0:
                if c + 1 < NCH:
                    start_pe(c + 1)
                pe_desc.pop(c).wait()
            gather_desc.pop(i).wait()
            if i + A < n_items:
                start_gather(i + A)

            buf, pe_buf = row_bufs[i % R], pe_bufs[c % 2]

            def add_row(r, carry, buf=buf, pe_buf=pe_buf):
                for dd in range(D // L):
                    sl = pl.ds(dd * L, L)
                    plsc.addupdate(buf.at[r, sl], pe_buf[r, sl])
                return carry

            lax.fori_loop(0, CH, add_row, None)
            store_desc[i] = pltpu.async_copy(
                buf, out_hbm.at[pl.ds(b * T + pbase + c * CH, CH)],
                s_sems[i % R])

        for j in sorted(store_desc):
            store_desc.pop(j).wait()

    out = sc_embed(xf, pe, table)
    return out.reshape(B, T, D)
